# baseline (speedup 1.0000x reference)
import functools

import jax
import jax.numpy as jnp
from jax import lax
from jax.experimental import pallas as pl
from jax.experimental.pallas import tpu as pltpu
from jax.experimental.pallas import tpu_sc as plsc

D_MODEL = 1024
SEQ_LEN = 8192
NUM_CORES = 2
NUM_SUBCORES = 16
NUM_WORKERS = NUM_CORES * NUM_SUBCORES
ROWS_PER_WORKER = SEQ_LEN // NUM_WORKERS
CHUNK = 16
NUM_CHUNKS = ROWS_PER_WORKER // CHUNK
NBUF = 7
LOOKAHEAD = 5


def _embed_body(idx_hbm, table_hbm, out_hbm, idx_v, *rest):
    bufs = rest[:NBUF]
    gsems = rest[NBUF:2 * NBUF]
    ssems = rest[2 * NBUF:]
    wid = lax.axis_index("s") * NUM_CORES + lax.axis_index("c")
    base = wid * ROWS_PER_WORKER
    pltpu.sync_copy(idx_hbm.at[pl.ds(base, ROWS_PER_WORKER)], idx_v)
    ghandles = [None] * NBUF
    shandles = [None] * NBUF
    for c in range(min(LOOKAHEAD, NUM_CHUNKS)):
        b = c % NBUF
        ghandles[b] = pltpu.async_copy(
            table_hbm.at[idx_v.at[pl.ds(c * CHUNK, CHUNK)]], bufs[b], gsems[b])
    for c in range(NUM_CHUNKS):
        b = c % NBUF
        g = c + LOOKAHEAD
        if g < NUM_CHUNKS:
            gb = g % NBUF
            if shandles[gb] is not None:
                shandles[gb].wait()
            ghandles[gb] = pltpu.async_copy(
                table_hbm.at[idx_v.at[pl.ds(g * CHUNK, CHUNK)]], bufs[gb], gsems[gb])
        ghandles[b].wait()
        shandles[b] = pltpu.async_copy(
            bufs[b], out_hbm.at[pl.ds(base + c * CHUNK, CHUNK)], ssems[b])
    for c in range(max(0, NUM_CHUNKS - NBUF), NUM_CHUNKS):
        b = c % NBUF
        if shandles[b] is not None:
            shandles[b].wait()
            shandles[b] = None


_embed_gather = functools.partial(
    pl.kernel,
    mesh=plsc.VectorSubcoreMesh(core_axis_name="c", subcore_axis_name="s"),
    out_type=jax.ShapeDtypeStruct((SEQ_LEN, D_MODEL), jnp.float32),
    scratch_types=(
        [pltpu.VMEM((ROWS_PER_WORKER,), jnp.int32)]
        + [pltpu.VMEM((CHUNK, D_MODEL), jnp.float32) for _ in range(NBUF)]
        + [pltpu.SemaphoreType.DMA for _ in range(2 * NBUF)]
    ),
)(_embed_body)


def kernel(positions, weight):
    return _embed_gather(positions.astype(jnp.int32), weight)
